# Initial kernel scaffold; baseline (speedup 1.0000x reference)
import jax
import jax.numpy as jnp
from jax.experimental import pallas as pl


def kernel(head_ids, tail_ids, head_value, tail_value, rc_ids, nd, prop_embed, fc2_W, fc2_b, lin1_W, lin1_b):
    raise NotImplementedError("write your pallas kernel here")


# speedup vs baseline: 3.1837x; 3.1837x over previous
import functools

import jax
import jax.numpy as jnp
from jax import lax
from jax.experimental import pallas as pl
from jax.experimental.pallas import tpu as pltpu
from jax.experimental.pallas import tpu_sc as plsc

B, L, D, H, R = 128, 200, 768, 8, 8
HP = 16
TW = R * HP

_VB = 2000


def _proj_body(prop_ref, w_ref, out_ref):
    out_ref[...] = jnp.dot(prop_ref[...], w_ref[...],
                           preferred_element_type=jnp.float32,
                           precision=lax.Precision.HIGHEST)


def _project_table(prop_embed, w_pad):
    V = prop_embed.shape[0]
    return pl.pallas_call(
        _proj_body,
        grid=(V // _VB,),
        in_specs=[
            pl.BlockSpec((_VB, D), lambda i: (i, 0)),
            pl.BlockSpec((D, TW), lambda i: (0, 0)),
        ],
        out_specs=pl.BlockSpec((_VB, TW), lambda i: (i, 0)),
        out_shape=jax.ShapeDtypeStruct((V, TW), jnp.float32),
    )(prop_embed, w_pad)


_NC, _NS = 2, 16
_NW = _NC * _NS
_NIDX = 2 * B * L
_BPW = _NIDX // _NW
_CH = 400


def _sc_gather(table, idx):
    mesh = plsc.VectorSubcoreMesh(core_axis_name="c", subcore_axis_name="s")

    @functools.partial(
        pl.kernel,
        mesh=mesh,
        out_type=jax.ShapeDtypeStruct((_NIDX, TW), jnp.float32),
        scratch_types=[
            pltpu.VMEM((_BPW,), jnp.int32),
            pltpu.VMEM((_CH, TW), jnp.float32),
            pltpu.VMEM((_CH, TW), jnp.float32),
            pltpu.SemaphoreType.DMA,
            pltpu.SemaphoreType.DMA,
        ],
    )
    def k(table_hbm, idx_hbm, out_hbm, idx_v, rows0, rows1, sem0, sem1):
        wid = lax.axis_index("s") * _NC + lax.axis_index("c")
        base = wid * _BPW
        pltpu.sync_copy(idx_hbm.at[pl.ds(base, _BPW)], idx_v)
        rows = (rows0, rows1)
        sems = (sem0, sem1)
        nch = _BPW // _CH
        copies = [None] * nch
        for c in range(nch):
            copies[c] = pltpu.async_copy(
                table_hbm.at[idx_v.at[pl.ds(c * _CH, _CH)]],
                rows[c % 2], sems[c % 2])
            if c >= 1:
                copies[c - 1].wait()
                pltpu.sync_copy(rows[(c - 1) % 2],
                                out_hbm.at[pl.ds(base + (c - 1) * _CH, _CH)])
        copies[nch - 1].wait()
        pltpu.sync_copy(rows[(nch - 1) % 2],
                        out_hbm.at[pl.ds(base + (nch - 1) * _CH, _CH)])

    return k(table, idx)


_BB = 8


def _attend_body(rc_ref, gh_ref, gt_ref, hv_ref, tv_ref, oh_ref, bq_ref,
                 wl_ref, bl_ref, ah_ref, at_ref, sc_ref):
    i = pl.program_id(0)
    onehot = oh_ref[...]
    bq = jnp.dot(onehot, bq_ref[...], preferred_element_type=jnp.float32)
    wl = jnp.dot(onehot, wl_ref[...], preferred_element_type=jnp.float32)
    bl = jnp.dot(onehot, bl_ref[...], preferred_element_type=jnp.float32)

    io_j = lax.broadcasted_iota(jnp.int32, (TW, HP), 0)
    io_h = lax.broadcasted_iota(jnp.int32, (TW, HP), 1)

    def select(g_ref):
        qs = []
        for b in range(_BB):
            rcb = rc_ref[i * _BB + b]
            sel = (io_j == rcb * HP + io_h).astype(jnp.float32)
            qs.append(jnp.dot(g_ref[b], sel,
                              preferred_element_type=jnp.float32).reshape(1, L, HP))
        return jnp.concatenate(qs, axis=0)

    def side(g_ref, v_ref, a_ref):
        q = jnp.maximum(select(g_ref) + bq[:, None, :], 0.0)
        m = jnp.max(q, axis=1, keepdims=True)
        e = jnp.exp(q - m)
        s = jnp.sum(e, axis=1, keepdims=True)
        wgt = e / s
        att = jnp.sum(wgt[:, :, :H], axis=2) * (1.0 / H)
        a_ref[...] = att
        return jnp.sum(v_ref[...] * att[:, :, None], axis=1)

    h_out = side(gh_ref, hv_ref, ah_ref)
    t_out = side(gt_ref, tv_ref, at_ref)
    score = (jnp.sum(h_out * wl[:, :D], axis=1)
             + jnp.sum(t_out * wl[:, D:], axis=1) + bl[:, 0])
    sc_ref[...] = score.reshape(1, 1, _BB)


def _attend(rc, g2, hv, tv, onehot, bq_pad, lin1_W, lin1_b2):
    nb = B // _BB
    return pl.pallas_call(
        _attend_body,
        grid=(nb,),
        in_specs=[
            pl.BlockSpec(memory_space=pltpu.SMEM),
            pl.BlockSpec((_BB, L, TW), lambda i: (i, 0, 0)),
            pl.BlockSpec((_BB, L, TW), lambda i: (i + nb, 0, 0)),
            pl.BlockSpec((_BB, L, D), lambda i: (i, 0, 0)),
            pl.BlockSpec((_BB, L, D), lambda i: (i, 0, 0)),
            pl.BlockSpec((_BB, R), lambda i: (i, 0)),
            pl.BlockSpec((R, HP), lambda i: (0, 0)),
            pl.BlockSpec((R, 2 * D), lambda i: (0, 0)),
            pl.BlockSpec((R, 1), lambda i: (0, 0)),
        ],
        out_specs=[
            pl.BlockSpec((_BB, L), lambda i: (i, 0)),
            pl.BlockSpec((_BB, L), lambda i: (i, 0)),
            pl.BlockSpec((1, 1, _BB), lambda i: (i, 0, 0)),
        ],
        out_shape=[
            jax.ShapeDtypeStruct((B, L), jnp.float32),
            jax.ShapeDtypeStruct((B, L), jnp.float32),
            jax.ShapeDtypeStruct((B // _BB, 1, _BB), jnp.float32),
        ],
    )(rc, g2, g2, hv, tv, onehot, bq_pad, lin1_W, lin1_b2)


def _loss_body(s_ref, nd_ref, out_ref):
    s = s_ref[...]
    nd = nd_ref[...]
    io = lax.broadcasted_iota(jnp.int32, (1, B), 1)
    big = jnp.int32(2 ** 30)
    pi = jnp.min(jnp.where(nd == 1, io, big))
    pi = jnp.where(pi == big, 0, pi)
    ni = jnp.min(jnp.where(nd == 0, io, big))
    ni = jnp.where(ni == big, 0, ni)
    pos = jnp.sum(jnp.where(io == pi, s, 0.0))
    neg = jnp.sum(jnp.where(io == ni, s, 0.0))
    out_ref[0, 0] = jnp.maximum(neg - pos + 1.0, 0.0)


def _loss(score_2d, nd_2d):
    return pl.pallas_call(
        _loss_body,
        out_specs=pl.BlockSpec(memory_space=pltpu.SMEM),
        out_shape=jax.ShapeDtypeStruct((1, 1), jnp.float32),
    )(score_2d, nd_2d)


def kernel(head_ids, tail_ids, head_value, tail_value, rc_ids, nd,
           prop_embed, fc2_W, fc2_b, lin1_W, lin1_b):
    w = jnp.transpose(fc2_W, (1, 0, 2))
    w = jnp.pad(w, ((0, 0), (0, HP - H), (0, 0)))
    w_pad = w.reshape(TW, D).T

    table = _project_table(prop_embed, w_pad)

    idx = jnp.concatenate([
        head_ids.astype(jnp.int32).reshape(-1),
        tail_ids.astype(jnp.int32).reshape(-1),
    ])
    g = _sc_gather(table, idx)
    g2 = g.reshape(2 * B, L, TW)

    rc = rc_ids.astype(jnp.int32)
    onehot = (rc[:, None] == jnp.arange(R, dtype=jnp.int32)[None, :]).astype(jnp.float32)
    bq_pad = jnp.pad(fc2_b.T, ((0, 0), (0, HP - H)))
    lin1_b2 = lin1_b.reshape(R, 1)

    att_h, att_t, score3 = _attend(rc, g2, head_value, tail_value,
                                   onehot, bq_pad, lin1_W, lin1_b2)
    total_score = score3.reshape(B)
    loss = _loss(total_score.reshape(1, B),
                 nd.reshape(1, B).astype(jnp.int32)).reshape(())
    return total_score, loss, att_h, att_t


# speedup vs baseline: 4.5066x; 1.4155x over previous
import functools

import jax
import jax.numpy as jnp
from jax import lax
from jax.experimental import pallas as pl
from jax.experimental.pallas import tpu as pltpu
from jax.experimental.pallas import tpu_sc as plsc

B, L, D, H, R = 128, 200, 768, 8, 8
HP = 16
TW = R * HP

_VB = 2000


def _proj_body(prop_ref, w_ref, out_ref):
    a = prop_ref[...]
    w = w_ref[...]
    ah = a.astype(jnp.bfloat16)
    al = (a - ah.astype(jnp.float32)).astype(jnp.bfloat16)
    wh = w.astype(jnp.bfloat16)
    wl = (w - wh.astype(jnp.float32)).astype(jnp.bfloat16)

    def mm(x, y):
        return jnp.dot(x, y, preferred_element_type=jnp.float32)

    out_ref[...] = mm(ah, wh) + mm(ah, wl) + mm(al, wh)


def _project_table(prop_embed, w_pad):
    V = prop_embed.shape[0]
    return pl.pallas_call(
        _proj_body,
        grid=(V // _VB,),
        in_specs=[
            pl.BlockSpec((_VB, D), lambda i: (i, 0)),
            pl.BlockSpec((D, TW), lambda i: (0, 0)),
        ],
        out_specs=pl.BlockSpec((_VB, TW), lambda i: (i, 0)),
        out_shape=jax.ShapeDtypeStruct((V, TW), jnp.float32),
    )(prop_embed, w_pad)


_NC, _NS = 2, 16
_NW = _NC * _NS
_NIDX = 2 * B * L
_BPW = _NIDX // _NW
_CH = 400


def _sc_gather(table, idx):
    mesh = plsc.VectorSubcoreMesh(core_axis_name="c", subcore_axis_name="s")

    @functools.partial(
        pl.kernel,
        mesh=mesh,
        out_type=jax.ShapeDtypeStruct((_NIDX, TW), jnp.float32),
        scratch_types=[
            pltpu.VMEM((_BPW,), jnp.int32),
            pltpu.VMEM((_CH, TW), jnp.float32),
            pltpu.VMEM((_CH, TW), jnp.float32),
            pltpu.SemaphoreType.DMA,
            pltpu.SemaphoreType.DMA,
        ],
    )
    def k(table_hbm, idx_hbm, out_hbm, idx_v, rows0, rows1, sem0, sem1):
        wid = lax.axis_index("s") * _NC + lax.axis_index("c")
        base = wid * _BPW
        pltpu.sync_copy(idx_hbm.at[pl.ds(base, _BPW)], idx_v)
        rows = (rows0, rows1)
        sems = (sem0, sem1)
        nch = _BPW // _CH
        copies = [None] * nch
        for c in range(nch):
            copies[c] = pltpu.async_copy(
                table_hbm.at[idx_v.at[pl.ds(c * _CH, _CH)]],
                rows[c % 2], sems[c % 2])
            if c >= 1:
                copies[c - 1].wait()
                pltpu.sync_copy(rows[(c - 1) % 2],
                                out_hbm.at[pl.ds(base + (c - 1) * _CH, _CH)])
        copies[nch - 1].wait()
        pltpu.sync_copy(rows[(nch - 1) % 2],
                        out_hbm.at[pl.ds(base + (nch - 1) * _CH, _CH)])

    return k(table, idx)


_BB = 8


def _attend_body(rc_ref, gh_ref, gt_ref, hv_ref, tv_ref, oh_ref, bq_ref,
                 wl_ref, bl_ref, ah_ref, at_ref, sc_ref):
    i = pl.program_id(0)
    onehot = oh_ref[...]
    bq = jnp.dot(onehot, bq_ref[...], preferred_element_type=jnp.float32)
    wl = jnp.dot(onehot, wl_ref[...], preferred_element_type=jnp.float32)
    bl = jnp.dot(onehot, bl_ref[...], preferred_element_type=jnp.float32)

    io_j = lax.broadcasted_iota(jnp.int32, (TW, HP), 0)
    io_h = lax.broadcasted_iota(jnp.int32, (TW, HP), 1)

    def select(g_ref):
        qs = []
        for b in range(_BB):
            rcb = rc_ref[i * _BB + b]
            sel = (io_j == rcb * HP + io_h).astype(jnp.float32)
            qs.append(jnp.dot(g_ref[b], sel,
                              preferred_element_type=jnp.float32).reshape(1, L, HP))
        return jnp.concatenate(qs, axis=0)

    def side(g_ref, v_ref, a_ref):
        q = jnp.maximum(select(g_ref) + bq[:, None, :], 0.0)
        m = jnp.max(q, axis=1, keepdims=True)
        e = jnp.exp(q - m)
        s = jnp.sum(e, axis=1, keepdims=True)
        wgt = e / s
        att = jnp.sum(wgt[:, :, :H], axis=2) * (1.0 / H)
        a_ref[...] = att
        return jnp.sum(v_ref[...] * att[:, :, None], axis=1)

    h_out = side(gh_ref, hv_ref, ah_ref)
    t_out = side(gt_ref, tv_ref, at_ref)
    score = (jnp.sum(h_out * wl[:, :D], axis=1)
             + jnp.sum(t_out * wl[:, D:], axis=1) + bl[:, 0])
    sc_ref[...] = score.reshape(1, 1, _BB)


def _attend(rc, g2, hv, tv, onehot, bq_pad, lin1_W, lin1_b2):
    nb = B // _BB
    return pl.pallas_call(
        _attend_body,
        grid=(nb,),
        in_specs=[
            pl.BlockSpec(memory_space=pltpu.SMEM),
            pl.BlockSpec((_BB, L, TW), lambda i: (i, 0, 0)),
            pl.BlockSpec((_BB, L, TW), lambda i: (i + nb, 0, 0)),
            pl.BlockSpec((_BB, L, D), lambda i: (i, 0, 0)),
            pl.BlockSpec((_BB, L, D), lambda i: (i, 0, 0)),
            pl.BlockSpec((_BB, R), lambda i: (i, 0)),
            pl.BlockSpec((R, HP), lambda i: (0, 0)),
            pl.BlockSpec((R, 2 * D), lambda i: (0, 0)),
            pl.BlockSpec((R, 1), lambda i: (0, 0)),
        ],
        out_specs=[
            pl.BlockSpec((_BB, L), lambda i: (i, 0)),
            pl.BlockSpec((_BB, L), lambda i: (i, 0)),
            pl.BlockSpec((1, 1, _BB), lambda i: (i, 0, 0)),
        ],
        out_shape=[
            jax.ShapeDtypeStruct((B, L), jnp.float32),
            jax.ShapeDtypeStruct((B, L), jnp.float32),
            jax.ShapeDtypeStruct((B // _BB, 1, _BB), jnp.float32),
        ],
    )(rc, g2, g2, hv, tv, onehot, bq_pad, lin1_W, lin1_b2)


def _loss_body(s_ref, nd_ref, out_ref):
    s = s_ref[...]
    nd = nd_ref[...]
    io = lax.broadcasted_iota(jnp.int32, (1, B), 1)
    big = jnp.int32(2 ** 30)
    pi = jnp.min(jnp.where(nd == 1, io, big))
    pi = jnp.where(pi == big, 0, pi)
    ni = jnp.min(jnp.where(nd == 0, io, big))
    ni = jnp.where(ni == big, 0, ni)
    pos = jnp.sum(jnp.where(io == pi, s, 0.0))
    neg = jnp.sum(jnp.where(io == ni, s, 0.0))
    out_ref[0, 0] = jnp.maximum(neg - pos + 1.0, 0.0)


def _loss(score_2d, nd_2d):
    return pl.pallas_call(
        _loss_body,
        out_specs=pl.BlockSpec(memory_space=pltpu.SMEM),
        out_shape=jax.ShapeDtypeStruct((1, 1), jnp.float32),
    )(score_2d, nd_2d)


def kernel(head_ids, tail_ids, head_value, tail_value, rc_ids, nd,
           prop_embed, fc2_W, fc2_b, lin1_W, lin1_b):
    w = jnp.transpose(fc2_W, (1, 0, 2))
    w = jnp.pad(w, ((0, 0), (0, HP - H), (0, 0)))
    w_pad = w.reshape(TW, D).T

    table = _project_table(prop_embed, w_pad)

    idx = jnp.concatenate([
        head_ids.astype(jnp.int32).reshape(-1),
        tail_ids.astype(jnp.int32).reshape(-1),
    ])
    g = _sc_gather(table, idx)
    g2 = g.reshape(2 * B, L, TW)

    rc = rc_ids.astype(jnp.int32)
    onehot = (rc[:, None] == jnp.arange(R, dtype=jnp.int32)[None, :]).astype(jnp.float32)
    bq_pad = jnp.pad(fc2_b.T, ((0, 0), (0, HP - H)))
    lin1_b2 = lin1_b.reshape(R, 1)

    att_h, att_t, score3 = _attend(rc, g2, head_value, tail_value,
                                   onehot, bq_pad, lin1_W, lin1_b2)
    total_score = score3.reshape(B)
    loss = _loss(total_score.reshape(1, B),
                 nd.reshape(1, B).astype(jnp.int32)).reshape(())
    return total_score, loss, att_h, att_t


# speedup vs baseline: 4.8692x; 1.0805x over previous
import functools

import jax
import jax.numpy as jnp
from jax import lax
from jax.experimental import pallas as pl
from jax.experimental.pallas import tpu as pltpu
from jax.experimental.pallas import tpu_sc as plsc

B, L, D, H, R = 128, 200, 768, 8, 8
HP = 16
TW = R * HP

_VB = 2000


def _proj_body(prop_ref, w_ref, out_ref):
    a = prop_ref[...]
    w = w_ref[...]
    ah = a.astype(jnp.bfloat16)
    al = (a - ah.astype(jnp.float32)).astype(jnp.bfloat16)
    wh = w.astype(jnp.bfloat16)

    def mm(x, y):
        return jnp.dot(x, y, preferred_element_type=jnp.float32)

    out_ref[...] = mm(ah, wh) + mm(al, wh)


def _project_table(prop_embed, w_pad):
    V = prop_embed.shape[0]
    return pl.pallas_call(
        _proj_body,
        grid=(V // _VB,),
        in_specs=[
            pl.BlockSpec((_VB, D), lambda i: (i, 0)),
            pl.BlockSpec((D, TW), lambda i: (0, 0)),
        ],
        out_specs=pl.BlockSpec((_VB, TW), lambda i: (i, 0)),
        out_shape=jax.ShapeDtypeStruct((V, TW), jnp.float32),
    )(prop_embed, w_pad)


_NC, _NS = 2, 16
_NW = _NC * _NS
_NIDX = 2 * B * L
_BPW = _NIDX // _NW
_CH = 400


def _sc_gather(table, idx):
    mesh = plsc.VectorSubcoreMesh(core_axis_name="c", subcore_axis_name="s")

    @functools.partial(
        pl.kernel,
        mesh=mesh,
        out_type=jax.ShapeDtypeStruct((_NIDX, TW), jnp.float32),
        scratch_types=[
            pltpu.VMEM((_BPW,), jnp.int32),
            pltpu.VMEM((_CH, TW), jnp.float32),
            pltpu.VMEM((_CH, TW), jnp.float32),
            pltpu.SemaphoreType.DMA,
            pltpu.SemaphoreType.DMA,
        ],
    )
    def k(table_hbm, idx_hbm, out_hbm, idx_v, rows0, rows1, sem0, sem1):
        wid = lax.axis_index("s") * _NC + lax.axis_index("c")
        base = wid * _BPW
        pltpu.sync_copy(idx_hbm.at[pl.ds(base, _BPW)], idx_v)
        rows = (rows0, rows1)
        sems = (sem0, sem1)
        nch = _BPW // _CH
        copies = [None] * nch
        for c in range(nch):
            copies[c] = pltpu.async_copy(
                table_hbm.at[idx_v.at[pl.ds(c * _CH, _CH)]],
                rows[c % 2], sems[c % 2])
            if c >= 1:
                copies[c - 1].wait()
                pltpu.sync_copy(rows[(c - 1) % 2],
                                out_hbm.at[pl.ds(base + (c - 1) * _CH, _CH)])
        copies[nch - 1].wait()
        pltpu.sync_copy(rows[(nch - 1) % 2],
                        out_hbm.at[pl.ds(base + (nch - 1) * _CH, _CH)])

    return k(table, idx)


_BB = 8


def _attend_body(rc_ref, gh_ref, gt_ref, hv_ref, tv_ref, oh_ref, bq_ref,
                 wl_ref, bl_ref, ah_ref, at_ref, sc_ref):
    i = pl.program_id(0)
    onehot = oh_ref[...]
    bq = jnp.dot(onehot, bq_ref[...], preferred_element_type=jnp.float32)
    wl = jnp.dot(onehot, wl_ref[...], preferred_element_type=jnp.float32)
    bl = jnp.dot(onehot, bl_ref[...], preferred_element_type=jnp.float32)

    io_j = lax.broadcasted_iota(jnp.int32, (TW, HP), 0)
    io_h = lax.broadcasted_iota(jnp.int32, (TW, HP), 1)

    def select(g_ref):
        qs = []
        for b in range(_BB):
            rcb = rc_ref[i * _BB + b]
            sel = (io_j == rcb * HP + io_h).astype(jnp.float32)
            qs.append(jnp.dot(g_ref[b], sel,
                              preferred_element_type=jnp.float32).reshape(1, L, HP))
        return jnp.concatenate(qs, axis=0)

    def side(g_ref, v_ref, a_ref):
        q = jnp.maximum(select(g_ref) + bq[:, None, :], 0.0)
        m = jnp.max(q, axis=1, keepdims=True)
        e = jnp.exp(q - m)
        s = jnp.sum(e, axis=1, keepdims=True)
        wgt = e / s
        att = jnp.sum(wgt[:, :, :H], axis=2) * (1.0 / H)
        a_ref[...] = att
        return jnp.sum(v_ref[...] * att[:, :, None], axis=1)

    h_out = side(gh_ref, hv_ref, ah_ref)
    t_out = side(gt_ref, tv_ref, at_ref)
    score = (jnp.sum(h_out * wl[:, :D], axis=1)
             + jnp.sum(t_out * wl[:, D:], axis=1) + bl[:, 0])
    sc_ref[...] = score.reshape(1, 1, _BB)


def _attend(rc, g2, hv, tv, onehot, bq_pad, lin1_W, lin1_b2):
    nb = B // _BB
    return pl.pallas_call(
        _attend_body,
        grid=(nb,),
        in_specs=[
            pl.BlockSpec(memory_space=pltpu.SMEM),
            pl.BlockSpec((_BB, L, TW), lambda i: (i, 0, 0)),
            pl.BlockSpec((_BB, L, TW), lambda i: (i + nb, 0, 0)),
            pl.BlockSpec((_BB, L, D), lambda i: (i, 0, 0)),
            pl.BlockSpec((_BB, L, D), lambda i: (i, 0, 0)),
            pl.BlockSpec((_BB, R), lambda i: (i, 0)),
            pl.BlockSpec((R, HP), lambda i: (0, 0)),
            pl.BlockSpec((R, 2 * D), lambda i: (0, 0)),
            pl.BlockSpec((R, 1), lambda i: (0, 0)),
        ],
        out_specs=[
            pl.BlockSpec((_BB, L), lambda i: (i, 0)),
            pl.BlockSpec((_BB, L), lambda i: (i, 0)),
            pl.BlockSpec((1, 1, _BB), lambda i: (i, 0, 0)),
        ],
        out_shape=[
            jax.ShapeDtypeStruct((B, L), jnp.float32),
            jax.ShapeDtypeStruct((B, L), jnp.float32),
            jax.ShapeDtypeStruct((B // _BB, 1, _BB), jnp.float32),
        ],
    )(rc, g2, g2, hv, tv, onehot, bq_pad, lin1_W, lin1_b2)


def _loss_body(s_ref, nd_ref, out_ref):
    s = s_ref[...]
    nd = nd_ref[...]
    io = lax.broadcasted_iota(jnp.int32, (1, B), 1)
    big = jnp.int32(2 ** 30)
    pi = jnp.min(jnp.where(nd == 1, io, big))
    pi = jnp.where(pi == big, 0, pi)
    ni = jnp.min(jnp.where(nd == 0, io, big))
    ni = jnp.where(ni == big, 0, ni)
    pos = jnp.sum(jnp.where(io == pi, s, 0.0))
    neg = jnp.sum(jnp.where(io == ni, s, 0.0))
    out_ref[0, 0] = jnp.maximum(neg - pos + 1.0, 0.0)


def _loss(score_2d, nd_2d):
    return pl.pallas_call(
        _loss_body,
        out_specs=pl.BlockSpec(memory_space=pltpu.SMEM),
        out_shape=jax.ShapeDtypeStruct((1, 1), jnp.float32),
    )(score_2d, nd_2d)


def kernel(head_ids, tail_ids, head_value, tail_value, rc_ids, nd,
           prop_embed, fc2_W, fc2_b, lin1_W, lin1_b):
    w = jnp.transpose(fc2_W, (1, 0, 2))
    w = jnp.pad(w, ((0, 0), (0, HP - H), (0, 0)))
    w_pad = w.reshape(TW, D).T

    table = _project_table(prop_embed, w_pad)

    idx = jnp.concatenate([
        head_ids.astype(jnp.int32).reshape(-1),
        tail_ids.astype(jnp.int32).reshape(-1),
    ])
    g = _sc_gather(table, idx)
    g2 = g.reshape(2 * B, L, TW)

    rc = rc_ids.astype(jnp.int32)
    onehot = (rc[:, None] == jnp.arange(R, dtype=jnp.int32)[None, :]).astype(jnp.float32)
    bq_pad = jnp.pad(fc2_b.T, ((0, 0), (0, HP - H)))
    lin1_b2 = lin1_b.reshape(R, 1)

    att_h, att_t, score3 = _attend(rc, g2, head_value, tail_value,
                                   onehot, bq_pad, lin1_W, lin1_b2)
    total_score = score3.reshape(B)
    loss = _loss(total_score.reshape(1, B),
                 nd.reshape(1, B).astype(jnp.int32)).reshape(())
    return total_score, loss, att_h, att_t


# speedup vs baseline: 5.1897x; 1.0658x over previous
import functools

import jax
import jax.numpy as jnp
from jax import lax
from jax.experimental import pallas as pl
from jax.experimental.pallas import tpu as pltpu
from jax.experimental.pallas import tpu_sc as plsc

B, L, D, H, R = 128, 200, 768, 8, 8
HP = 16
TW = R * HP

_VB = 4000


def _proj_body(prop_ref, w_ref, out_ref):
    a = prop_ref[...]
    w = w_ref[...]
    ah = a.astype(jnp.bfloat16)
    al = (a - ah.astype(jnp.float32)).astype(jnp.bfloat16)
    wh = w.astype(jnp.bfloat16)

    def mm(x, y):
        return jnp.dot(x, y, preferred_element_type=jnp.float32)

    out_ref[...] = mm(ah, wh) + mm(al, wh)


def _project_table(prop_embed, w_pad):
    V = prop_embed.shape[0]
    return pl.pallas_call(
        _proj_body,
        grid=(V // _VB,),
        in_specs=[
            pl.BlockSpec((_VB, D), lambda i: (i, 0)),
            pl.BlockSpec((D, TW), lambda i: (0, 0)),
        ],
        out_specs=pl.BlockSpec((_VB, TW), lambda i: (i, 0)),
        out_shape=jax.ShapeDtypeStruct((V, TW), jnp.float32),
    )(prop_embed, w_pad)


_NC, _NS = 2, 16
_NW = _NC * _NS
_NIDX = 2 * B * L
_BPW = _NIDX // _NW
_CH = 400


def _sc_gather(table, idx):
    mesh = plsc.VectorSubcoreMesh(core_axis_name="c", subcore_axis_name="s")

    @functools.partial(
        pl.kernel,
        mesh=mesh,
        out_type=jax.ShapeDtypeStruct((_NIDX, TW), jnp.float32),
        scratch_types=[
            pltpu.VMEM((_BPW,), jnp.int32),
            pltpu.VMEM((_CH, TW), jnp.float32),
            pltpu.VMEM((_CH, TW), jnp.float32),
            pltpu.SemaphoreType.DMA,
            pltpu.SemaphoreType.DMA,
        ],
    )
    def k(table_hbm, idx_hbm, out_hbm, idx_v, rows0, rows1, sem0, sem1):
        wid = lax.axis_index("s") * _NC + lax.axis_index("c")
        base = wid * _BPW
        pltpu.sync_copy(idx_hbm.at[pl.ds(base, _BPW)], idx_v)
        rows = (rows0, rows1)
        sems = (sem0, sem1)
        nch = _BPW // _CH
        copies = [None] * nch
        for c in range(nch):
            copies[c] = pltpu.async_copy(
                table_hbm.at[idx_v.at[pl.ds(c * _CH, _CH)]],
                rows[c % 2], sems[c % 2])
            if c >= 1:
                copies[c - 1].wait()
                pltpu.sync_copy(rows[(c - 1) % 2],
                                out_hbm.at[pl.ds(base + (c - 1) * _CH, _CH)])
        copies[nch - 1].wait()
        pltpu.sync_copy(rows[(nch - 1) % 2],
                        out_hbm.at[pl.ds(base + (nch - 1) * _CH, _CH)])

    return k(table, idx)


_BB = 8


def _attend_body(rc_ref, gh_ref, gt_ref, hv_ref, tv_ref, oh_ref, bq_ref,
                 wl_ref, bl_ref, ah_ref, at_ref, sc_ref):
    i = pl.program_id(0)
    onehot = oh_ref[...]
    bq = jnp.dot(onehot, bq_ref[...], preferred_element_type=jnp.float32)
    wl = jnp.dot(onehot, wl_ref[...], preferred_element_type=jnp.float32)
    bl = jnp.dot(onehot, bl_ref[...], preferred_element_type=jnp.float32)

    io_j = lax.broadcasted_iota(jnp.int32, (TW, HP), 0)
    io_h = lax.broadcasted_iota(jnp.int32, (TW, HP), 1)

    def select(g_ref):
        qs = []
        for b in range(_BB):
            rcb = rc_ref[i * _BB + b]
            sel = (io_j == rcb * HP + io_h).astype(jnp.float32)
            qs.append(jnp.dot(g_ref[b], sel,
                              preferred_element_type=jnp.float32).reshape(1, L, HP))
        return jnp.concatenate(qs, axis=0)

    def side(g_ref, v_ref, a_ref):
        q = jnp.maximum(select(g_ref) + bq[:, None, :], 0.0)
        m = jnp.max(q, axis=1, keepdims=True)
        e = jnp.exp(q - m)
        s = jnp.sum(e, axis=1, keepdims=True)
        wgt = e / s
        att = jnp.sum(wgt[:, :, :H], axis=2) * (1.0 / H)
        a_ref[...] = att
        return jnp.sum(v_ref[...] * att[:, :, None], axis=1)

    h_out = side(gh_ref, hv_ref, ah_ref)
    t_out = side(gt_ref, tv_ref, at_ref)
    score = (jnp.sum(h_out * wl[:, :D], axis=1)
             + jnp.sum(t_out * wl[:, D:], axis=1) + bl[:, 0])
    sc_ref[...] = score.reshape(1, 1, _BB)


def _attend(rc, g2, hv, tv, onehot, bq_pad, lin1_W, lin1_b2):
    nb = B // _BB
    return pl.pallas_call(
        _attend_body,
        grid=(nb,),
        in_specs=[
            pl.BlockSpec(memory_space=pltpu.SMEM),
            pl.BlockSpec((_BB, L, TW), lambda i: (i, 0, 0)),
            pl.BlockSpec((_BB, L, TW), lambda i: (i + nb, 0, 0)),
            pl.BlockSpec((_BB, L, D), lambda i: (i, 0, 0)),
            pl.BlockSpec((_BB, L, D), lambda i: (i, 0, 0)),
            pl.BlockSpec((_BB, R), lambda i: (i, 0)),
            pl.BlockSpec((R, HP), lambda i: (0, 0)),
            pl.BlockSpec((R, 2 * D), lambda i: (0, 0)),
            pl.BlockSpec((R, 1), lambda i: (0, 0)),
        ],
        out_specs=[
            pl.BlockSpec((_BB, L), lambda i: (i, 0)),
            pl.BlockSpec((_BB, L), lambda i: (i, 0)),
            pl.BlockSpec((1, 1, _BB), lambda i: (i, 0, 0)),
        ],
        out_shape=[
            jax.ShapeDtypeStruct((B, L), jnp.float32),
            jax.ShapeDtypeStruct((B, L), jnp.float32),
            jax.ShapeDtypeStruct((B // _BB, 1, _BB), jnp.float32),
        ],
    )(rc, g2, g2, hv, tv, onehot, bq_pad, lin1_W, lin1_b2)


def _loss_body(s_ref, nd_ref, out_ref):
    s = s_ref[...]
    nd = nd_ref[...]
    io = lax.broadcasted_iota(jnp.int32, (1, B), 1)
    big = jnp.int32(2 ** 30)
    pi = jnp.min(jnp.where(nd == 1, io, big))
    pi = jnp.where(pi == big, 0, pi)
    ni = jnp.min(jnp.where(nd == 0, io, big))
    ni = jnp.where(ni == big, 0, ni)
    pos = jnp.sum(jnp.where(io == pi, s, 0.0))
    neg = jnp.sum(jnp.where(io == ni, s, 0.0))
    out_ref[0, 0] = jnp.maximum(neg - pos + 1.0, 0.0)


def _loss(score_2d, nd_2d):
    return pl.pallas_call(
        _loss_body,
        out_specs=pl.BlockSpec(memory_space=pltpu.SMEM),
        out_shape=jax.ShapeDtypeStruct((1, 1), jnp.float32),
    )(score_2d, nd_2d)


def kernel(head_ids, tail_ids, head_value, tail_value, rc_ids, nd,
           prop_embed, fc2_W, fc2_b, lin1_W, lin1_b):
    w = jnp.transpose(fc2_W, (1, 0, 2))
    w = jnp.pad(w, ((0, 0), (0, HP - H), (0, 0)))
    w_pad = w.reshape(TW, D).T

    table = _project_table(prop_embed, w_pad)

    idx = jnp.concatenate([
        head_ids.astype(jnp.int32).reshape(-1),
        tail_ids.astype(jnp.int32).reshape(-1),
    ])
    g = _sc_gather(table, idx)
    g2 = g.reshape(2 * B, L, TW)

    rc = rc_ids.astype(jnp.int32)
    onehot = (rc[:, None] == jnp.arange(R, dtype=jnp.int32)[None, :]).astype(jnp.float32)
    bq_pad = jnp.pad(fc2_b.T, ((0, 0), (0, HP - H)))
    lin1_b2 = lin1_b.reshape(R, 1)

    att_h, att_t, score3 = _attend(rc, g2, head_value, tail_value,
                                   onehot, bq_pad, lin1_W, lin1_b2)
    total_score = score3.reshape(B)
    loss = _loss(total_score.reshape(1, B),
                 nd.reshape(1, B).astype(jnp.int32)).reshape(())
    return total_score, loss, att_h, att_t


# speedup vs baseline: 5.2551x; 1.0126x over previous
import functools

import jax
import jax.numpy as jnp
from jax import lax
from jax.experimental import pallas as pl
from jax.experimental.pallas import tpu as pltpu
from jax.experimental.pallas import tpu_sc as plsc

B, L, D, H, R = 128, 200, 768, 8, 8
HP = 16
TW = R * HP

_VB = 5000


def _proj_body(prop_ref, w_ref, out_ref):
    a = prop_ref[...]
    w = w_ref[...]
    ah = a.astype(jnp.bfloat16)
    al = (a - ah.astype(jnp.float32)).astype(jnp.bfloat16)
    wh = w.astype(jnp.bfloat16)

    def mm(x, y):
        return jnp.dot(x, y, preferred_element_type=jnp.float32)

    out_ref[...] = mm(ah, wh) + mm(al, wh)


def _project_table(prop_embed, w_pad):
    V = prop_embed.shape[0]
    return pl.pallas_call(
        _proj_body,
        grid=(V // _VB,),
        in_specs=[
            pl.BlockSpec((_VB, D), lambda i: (i, 0)),
            pl.BlockSpec((D, TW), lambda i: (0, 0)),
        ],
        out_specs=pl.BlockSpec((_VB, TW), lambda i: (i, 0)),
        out_shape=jax.ShapeDtypeStruct((V, TW), jnp.float32),
    )(prop_embed, w_pad)


_NC, _NS = 2, 16
_NW = _NC * _NS
_NIDX = 2 * B * L
_BPW = _NIDX // _NW
_CH = 400


def _sc_gather(table, idx):
    mesh = plsc.VectorSubcoreMesh(core_axis_name="c", subcore_axis_name="s")

    @functools.partial(
        pl.kernel,
        mesh=mesh,
        out_type=jax.ShapeDtypeStruct((_NIDX, TW), jnp.float32),
        scratch_types=[
            pltpu.VMEM((_BPW,), jnp.int32),
            pltpu.VMEM((_CH, TW), jnp.float32),
            pltpu.VMEM((_CH, TW), jnp.float32),
            pltpu.SemaphoreType.DMA,
            pltpu.SemaphoreType.DMA,
        ],
    )
    def k(table_hbm, idx_hbm, out_hbm, idx_v, rows0, rows1, sem0, sem1):
        wid = lax.axis_index("s") * _NC + lax.axis_index("c")
        base = wid * _BPW
        pltpu.sync_copy(idx_hbm.at[pl.ds(base, _BPW)], idx_v)
        rows = (rows0, rows1)
        sems = (sem0, sem1)
        nch = _BPW // _CH
        copies = [None] * nch
        for c in range(nch):
            copies[c] = pltpu.async_copy(
                table_hbm.at[idx_v.at[pl.ds(c * _CH, _CH)]],
                rows[c % 2], sems[c % 2])
            if c >= 1:
                copies[c - 1].wait()
                pltpu.sync_copy(rows[(c - 1) % 2],
                                out_hbm.at[pl.ds(base + (c - 1) * _CH, _CH)])
        copies[nch - 1].wait()
        pltpu.sync_copy(rows[(nch - 1) % 2],
                        out_hbm.at[pl.ds(base + (nch - 1) * _CH, _CH)])

    return k(table, idx)


_BB = 8


def _attend_body(rc_ref, gh_ref, gt_ref, hv_ref, tv_ref, oh_ref, bq_ref,
                 wl_ref, bl_ref, ah_ref, at_ref, sc_ref):
    i = pl.program_id(0)
    onehot = oh_ref[...]
    bq = jnp.dot(onehot, bq_ref[...], preferred_element_type=jnp.float32)
    wl = jnp.dot(onehot, wl_ref[...], preferred_element_type=jnp.float32)
    bl = jnp.dot(onehot, bl_ref[...], preferred_element_type=jnp.float32)

    io_j = lax.broadcasted_iota(jnp.int32, (TW, HP), 0)
    io_h = lax.broadcasted_iota(jnp.int32, (TW, HP), 1)

    def select(g_ref):
        qs = []
        for b in range(_BB):
            rcb = rc_ref[i * _BB + b]
            sel = (io_j == rcb * HP + io_h).astype(jnp.float32)
            qs.append(jnp.dot(g_ref[b], sel,
                              preferred_element_type=jnp.float32).reshape(1, L, HP))
        return jnp.concatenate(qs, axis=0)

    def side(g_ref, v_ref, a_ref):
        q = jnp.maximum(select(g_ref) + bq[:, None, :], 0.0)
        m = jnp.max(q, axis=1, keepdims=True)
        e = jnp.exp(q - m)
        s = jnp.sum(e, axis=1, keepdims=True)
        wgt = e / s
        att = jnp.sum(wgt[:, :, :H], axis=2) * (1.0 / H)
        a_ref[...] = att
        return jnp.sum(v_ref[...] * att[:, :, None], axis=1)

    h_out = side(gh_ref, hv_ref, ah_ref)
    t_out = side(gt_ref, tv_ref, at_ref)
    score = (jnp.sum(h_out * wl[:, :D], axis=1)
             + jnp.sum(t_out * wl[:, D:], axis=1) + bl[:, 0])
    sc_ref[...] = score.reshape(1, 1, _BB)


def _attend(rc, g2, hv, tv, onehot, bq_pad, lin1_W, lin1_b2):
    nb = B // _BB
    return pl.pallas_call(
        _attend_body,
        grid=(nb,),
        in_specs=[
            pl.BlockSpec(memory_space=pltpu.SMEM),
            pl.BlockSpec((_BB, L, TW), lambda i: (i, 0, 0)),
            pl.BlockSpec((_BB, L, TW), lambda i: (i + nb, 0, 0)),
            pl.BlockSpec((_BB, L, D), lambda i: (i, 0, 0)),
            pl.BlockSpec((_BB, L, D), lambda i: (i, 0, 0)),
            pl.BlockSpec((_BB, R), lambda i: (i, 0)),
            pl.BlockSpec((R, HP), lambda i: (0, 0)),
            pl.BlockSpec((R, 2 * D), lambda i: (0, 0)),
            pl.BlockSpec((R, 1), lambda i: (0, 0)),
        ],
        out_specs=[
            pl.BlockSpec((_BB, L), lambda i: (i, 0)),
            pl.BlockSpec((_BB, L), lambda i: (i, 0)),
            pl.BlockSpec((1, 1, _BB), lambda i: (i, 0, 0)),
        ],
        out_shape=[
            jax.ShapeDtypeStruct((B, L), jnp.float32),
            jax.ShapeDtypeStruct((B, L), jnp.float32),
            jax.ShapeDtypeStruct((B // _BB, 1, _BB), jnp.float32),
        ],
    )(rc, g2, g2, hv, tv, onehot, bq_pad, lin1_W, lin1_b2)


def _loss_body(s_ref, nd_ref, out_ref):
    s = s_ref[...]
    nd = nd_ref[...]
    io = lax.broadcasted_iota(jnp.int32, (1, B), 1)
    big = jnp.int32(2 ** 30)
    pi = jnp.min(jnp.where(nd == 1, io, big))
    pi = jnp.where(pi == big, 0, pi)
    ni = jnp.min(jnp.where(nd == 0, io, big))
    ni = jnp.where(ni == big, 0, ni)
    pos = jnp.sum(jnp.where(io == pi, s, 0.0))
    neg = jnp.sum(jnp.where(io == ni, s, 0.0))
    out_ref[0, 0] = jnp.maximum(neg - pos + 1.0, 0.0)


def _loss(score_2d, nd_2d):
    return pl.pallas_call(
        _loss_body,
        out_specs=pl.BlockSpec(memory_space=pltpu.SMEM),
        out_shape=jax.ShapeDtypeStruct((1, 1), jnp.float32),
    )(score_2d, nd_2d)


def kernel(head_ids, tail_ids, head_value, tail_value, rc_ids, nd,
           prop_embed, fc2_W, fc2_b, lin1_W, lin1_b):
    w = jnp.transpose(fc2_W, (1, 0, 2))
    w = jnp.pad(w, ((0, 0), (0, HP - H), (0, 0)))
    w_pad = w.reshape(TW, D).T

    table = _project_table(prop_embed, w_pad)

    idx = jnp.concatenate([
        head_ids.astype(jnp.int32).reshape(-1),
        tail_ids.astype(jnp.int32).reshape(-1),
    ])
    g = _sc_gather(table, idx)
    g2 = g.reshape(2 * B, L, TW)

    rc = rc_ids.astype(jnp.int32)
    onehot = (rc[:, None] == jnp.arange(R, dtype=jnp.int32)[None, :]).astype(jnp.float32)
    bq_pad = jnp.pad(fc2_b.T, ((0, 0), (0, HP - H)))
    lin1_b2 = lin1_b.reshape(R, 1)

    att_h, att_t, score3 = _attend(rc, g2, head_value, tail_value,
                                   onehot, bq_pad, lin1_W, lin1_b2)
    total_score = score3.reshape(B)
    loss = _loss(total_score.reshape(1, B),
                 nd.reshape(1, B).astype(jnp.int32)).reshape(())
    return total_score, loss, att_h, att_t
